# Initial kernel scaffold; baseline (speedup 1.0000x reference)
import jax
import jax.numpy as jnp
from jax.experimental import pallas as pl


def kernel(x, edge_index, edge_weight, W, b, edge_mask1, feat_mask1, edge_mask2, feat_mask2):
    raise NotImplementedError("write your pallas kernel here")


# speedup vs baseline: 13.1863x; 13.1863x over previous
import jax
import jax.numpy as jnp
from jax import lax
from jax.experimental import pallas as pl
from jax.experimental.pallas import tpu as pltpu
from jax.experimental.pallas import tpu_sc as plsc

N = 10000
D_FEAT = 128
HIDDEN = 128
E = 320000

NC = 2
NS = 16
NW = NC * NS
NPAD = 10240
RW = 128
EP = 327680
NROWS = EP // RW

f32 = jnp.float32
i32 = jnp.int32


def _vmesh():
    return plsc.VectorSubcoreMesh(core_axis_name="c", subcore_axis_name="s")


def _zero_vec_ref(ref, nwords):
    def body(i, _):
        ref[pl.ds(i * 16, 16)] = jnp.zeros((16,), f32)
        return 0
    lax.fori_loop(0, nwords // 16, body, 0)


def _deg_body(src_h, dst_h, ew_h, m1_h, m2_h, out_h,
              d0s, d0d, d1s, d1d, d2s, d2d,
              srcb, dstb, ewb, m1b, m2b, ew1b, ew2b, zbuf, sem):
    cid = lax.axis_index("c")
    sid = lax.axis_index("s")

    _zero_vec_ref(zbuf, 640)
    for ref in (d0s, d0d, d1s, d1d, d2s, d2d):
        pltpu.sync_copy(zbuf, ref.at[pl.ds(sid * 640, 640)])
    plsc.subcore_barrier()

    wid = cid * NS + sid
    rows_per_tile = NROWS // NW
    CH = 16

    def chunk(ch, _):
        rb = wid * rows_per_tile + ch * CH
        pltpu.sync_copy(src_h.at[pl.ds(rb, CH), :], srcb)
        pltpu.sync_copy(dst_h.at[pl.ds(rb, CH), :], dstb)
        pltpu.sync_copy(ew_h.at[pl.ds(rb, CH), :], ewb)
        pltpu.sync_copy(m1_h.at[pl.ds(rb, CH), :], m1b)
        pltpu.sync_copy(m2_h.at[pl.ds(rb, CH), :], m2b)

        def cw(r, _):
            for q in range(RW // 16):
                e = ewb[r, pl.ds(q * 16, 16)]
                ew1b[r, pl.ds(q * 16, 16)] = e * m1b[r, pl.ds(q * 16, 16)]
                ew2b[r, pl.ds(q * 16, 16)] = e * m2b[r, pl.ds(q * 16, 16)]
            return 0
        lax.fori_loop(0, CH, cw, 0)

        for g in range(4):
            cps = []
            for rr in range(4):
                r = g * 4 + rr
                for val, idx, ref in ((ewb, srcb, d0s), (ewb, dstb, d0d),
                                      (ew1b, srcb, d1s), (ew1b, dstb, d1d),
                                      (ew2b, srcb, d2s), (ew2b, dstb, d2d)):
                    cps.append(pltpu.async_copy(
                        val.at[r], ref.at[idx.at[r]], sem, add=True))
            for cp in cps:
                cp.wait()
        return 0
    lax.fori_loop(0, rows_per_tile // CH, chunk, 0)

    plsc.subcore_barrier()

    @pl.when(sid < 8)
    def _():
        for k, ref in enumerate((d0s, d0d, d1s, d1d, d2s, d2d)):
            pltpu.sync_copy(
                ref.at[pl.ds(sid * 1280, 1280)],
                out_h.at[pl.ds(k * (NC * NPAD) + cid * NPAD + sid * 1280,
                               1280)])


def _deg_call(src2, dst2, ew2, m12, m22):
    fn = pl.kernel(
        _deg_body,
        out_type=jax.ShapeDtypeStruct((6 * NC * NPAD,), f32),
        mesh=_vmesh(),
        scratch_types=[pltpu.VMEM_SHARED((NPAD,), f32) for _ in range(6)]
        + [pltpu.VMEM((16, RW), i32) for _ in range(2)]
        + [pltpu.VMEM((16, RW), f32) for _ in range(5)]
        + [pltpu.VMEM((640,), f32), pltpu.SemaphoreType.DMA],
        compiler_params=pltpu.CompilerParams(use_tc_tiling_on_sc=False),
    )
    return fn(src2, dst2, ew2, m12, m22)


def _dinv_body(deg_ref, out_ref):
    d = deg_ref[:, 0, :] + deg_ref[:, 1, :]
    out_ref[...] = lax.rsqrt(jnp.maximum(d, 1e-12))


def _dinv_call(degpart):
    return pl.pallas_call(
        _dinv_body,
        out_shape=jax.ShapeDtypeStruct((6, NPAD), f32),
    )(degpart)


def _coef_body(src_h, dst_h, ew_h, m1_h, m2_h,
               h0s, h0d, h1s, h1d, h2s, h2d,
               coef_h, sp_h,
               s0, s1, s2,
               srcb, dstb, ewb, m1b, m2b, c0b, c1b, c2b,
               g0b, g1b, g2b, g3b, g4b, g5b, zbuf, sem):
    cid = lax.axis_index("c")
    sid = lax.axis_index("s")

    _zero_vec_ref(zbuf, 640)
    for ref in (s0, s1, s2):
        pltpu.sync_copy(zbuf, ref.at[pl.ds(sid * 640, 640)])
    plsc.subcore_barrier()

    wid = cid * NS + sid
    rows_per_tile = NROWS // NW
    CH = 16

    def chunk(ch, _):
        rb = wid * rows_per_tile + ch * CH
        pltpu.sync_copy(src_h.at[pl.ds(rb, CH), :], srcb)
        pltpu.sync_copy(dst_h.at[pl.ds(rb, CH), :], dstb)
        pltpu.sync_copy(ew_h.at[pl.ds(rb, CH), :], ewb)
        pltpu.sync_copy(m1_h.at[pl.ds(rb, CH), :], m1b)
        pltpu.sync_copy(m2_h.at[pl.ds(rb, CH), :], m2b)

        cps = []
        for r in range(CH):
            for tab, idx, dstv in ((h0s, srcb, g0b), (h0d, dstb, g1b),
                                   (h1s, srcb, g2b), (h1d, dstb, g3b),
                                   (h2s, srcb, g4b), (h2d, dstb, g5b)):
                cps.append(pltpu.async_copy(
                    tab.at[idx.at[r]], dstv.at[r], sem))
        for cp in cps:
            cp.wait()

        def crow(r, _):
            for q in range(RW // 16):
                sl = pl.ds(q * 16, 16)
                ew = ewb[r, sl]
                c0b[r, sl] = ew * g0b[r, sl] * g1b[r, sl]
                c1b[r, sl] = ew * m1b[r, sl] * g2b[r, sl] * g3b[r, sl]
                c2b[r, sl] = ew * m2b[r, sl] * g4b[r, sl] * g5b[r, sl]
            return 0
        lax.fori_loop(0, CH, crow, 0)

        pltpu.sync_copy(c0b, coef_h.at[0, pl.ds(rb, CH), :])
        pltpu.sync_copy(c1b, coef_h.at[1, pl.ds(rb, CH), :])
        pltpu.sync_copy(c2b, coef_h.at[2, pl.ds(rb, CH), :])

        for g in range(4):
            cps = []
            for rr in range(4):
                r = g * 4 + rr
                for val, ref in ((c0b, s0), (c1b, s1), (c2b, s2)):
                    cps.append(pltpu.async_copy(
                        val.at[r], ref.at[dstb.at[r]], sem, add=True))
            for cp in cps:
                cp.wait()
        return 0
    lax.fori_loop(0, rows_per_tile // CH, chunk, 0)

    plsc.subcore_barrier()

    @pl.when(sid < 8)
    def _():
        for k, ref in enumerate((s0, s1, s2)):
            pltpu.sync_copy(
                ref.at[pl.ds(sid * 1280, 1280)],
                sp_h.at[pl.ds(k * (NC * NPAD) + cid * NPAD + sid * 1280,
                              1280)])


def _coef_call(src2, dst2, ew2, m12, m22, dinv):
    fn = pl.kernel(
        _coef_body,
        out_type=(jax.ShapeDtypeStruct((3, NROWS, RW), f32),
                  jax.ShapeDtypeStruct((3 * NC * NPAD,), f32)),
        mesh=_vmesh(),
        scratch_types=[pltpu.VMEM_SHARED((NPAD,), f32) for _ in range(3)]
        + [pltpu.VMEM((16, RW), i32) for _ in range(2)]
        + [pltpu.VMEM((16, RW), f32) for _ in range(12)]
        + [pltpu.VMEM((640,), f32), pltpu.SemaphoreType.DMA],
        compiler_params=pltpu.CompilerParams(use_tc_tiling_on_sc=False),
    )
    return fn(src2, dst2, ew2, m12, m22,
              dinv[0], dinv[1], dinv[2], dinv[3], dinv[4], dinv[5])


def _make_agg_body(klist):
    KN = len(klist)
    KC = KN * 64

    def _agg_body(x0_h, x1_h, src_h, dst_h, coef_h, a_h,
                  acc, srcb, dstb, c0b, c1b, xr, stg, zbuf, semg, sems):
        cid = lax.axis_index("c")
        sid = lax.axis_index("s")
        cbs = (c0b, c1b)[:KN]

        def zb(i, _):
            for q in range(KC // 16):
                zbuf[i, pl.ds(q * 16, 16)] = jnp.zeros((16,), f32)
            return 0
        lax.fori_loop(0, 8, zb, 0)

        def zrow(i, _):
            pltpu.sync_copy(zbuf, acc.at[pl.ds(sid * 640 + i * 8, 8), :])
            return 0
        lax.fori_loop(0, 80, zrow, 0)
        plsc.subcore_barrier()

        rows_per_tile = NROWS // NS
        CH = 2

        def gather(r, off, slot):
            idx = srcb.at[r, pl.ds(off, 64)]

            @pl.when(cid == 0)
            def _():
                pltpu.async_copy(x0_h.at[idx], xr.at[slot], semg)

            @pl.when(cid == 1)
            def _():
                pltpu.async_copy(x1_h.at[idx], xr.at[slot], semg)
            return pltpu.make_async_copy(x0_h.at[idx], xr.at[slot], semg)

        def chunk(ch, _):
            rb = sid * rows_per_tile + ch * CH
            pltpu.sync_copy(src_h.at[pl.ds(rb, CH), :], srcb)
            pltpu.sync_copy(dst_h.at[pl.ds(rb, CH), :], dstb)
            for kk, cb in zip(klist, cbs):
                pltpu.sync_copy(coef_h.at[kk, pl.ds(rb, CH), :], cb)

            gw = [None] * 4
            scs = [None] * 4
            gw[0] = gather(0, 0, 0)
            for h in range(4):
                r, off, slot = h // 2, (h % 2) * 64, h % 2
                if h + 1 < 4:
                    r2, off2 = (h + 1) // 2, ((h + 1) % 2) * 64
                    gw[h + 1] = gather(r2, off2, (h + 1) % 2)
                gw[h].wait()
                if h >= 2:
                    scs[h - 2].wait()

                def egroup(g, _, r=r, off=off, slot=slot):
                    cvs = [cb[r, pl.ds(off + g * 16, 16)] for cb in cbs]
                    for li in range(16):
                        j = g * 16 + li
                        for q in range(4):
                            xq = xr[slot, j, pl.ds(q * 16, 16)]
                            for ki in range(KN):
                                stg[slot, j, pl.ds(ki * 64 + q * 16, 16)] = (
                                    cvs[ki][li] * xq)
                    return 0
                lax.fori_loop(0, 4, egroup, 0)
                scs[h] = pltpu.async_copy(
                    stg.at[slot], acc.at[dstb.at[r, pl.ds(off, 64)]],
                    sems, add=True)
            scs[2].wait()
            scs[3].wait()
            return 0
        lax.fori_loop(0, rows_per_tile // CH, chunk, 0)

        plsc.subcore_barrier()

        @pl.when(sid < 10)
        def _():
            for ki in range(KN):
                pltpu.sync_copy(
                    acc.at[pl.ds(sid * 1000, 1000), pl.ds(ki * 64, 64)],
                    a_h.at[cid, ki, pl.ds(sid * 1000, 1000), :])
    return _agg_body


def _agg_call(x0, x1, src2, dst2, coef, klist):
    KN = len(klist)
    KC = KN * 64
    fn = pl.kernel(
        _make_agg_body(klist),
        out_type=jax.ShapeDtypeStruct((NC, KN, N, 64), f32),
        mesh=_vmesh(),
        scratch_types=[pltpu.VMEM_SHARED((NPAD, KC), f32)]
        + [pltpu.VMEM((2, RW), i32) for _ in range(2)]
        + [pltpu.VMEM((2, RW), f32) for _ in range(2)]
        + [pltpu.VMEM((2, 64, 64), f32), pltpu.VMEM((2, 64, KC), f32)]
        + [pltpu.VMEM((8, KC), f32),
           pltpu.SemaphoreType.DMA, pltpu.SemaphoreType.DMA],
        compiler_params=pltpu.CompilerParams(use_tc_tiling_on_sc=False),
    )
    return fn(x0, x1, src2, dst2, coef)


def _mm_body(a_ref, w_ref, fm_ref, b_ref, s_ref, out_ref):
    am = a_ref[0] * fm_ref[0]
    h = jnp.dot(am, w_ref[...], preferred_element_type=f32,
                precision=lax.Precision.HIGHEST)
    sv = s_ref[0, :, 0] + s_ref[0, :, 1]
    out_ref[0] = jnp.maximum(h + sv[:, None] * b_ref[...], 0.0)


def _mm_call(A, W, b2, fmT, sT):
    RB = 2000
    return pl.pallas_call(
        _mm_body,
        grid=(3, N // RB),
        in_specs=[
            pl.BlockSpec((1, RB, D_FEAT), lambda k, i: (k, i, 0)),
            pl.BlockSpec((D_FEAT, HIDDEN), lambda k, i: (0, 0)),
            pl.BlockSpec((1, 1, D_FEAT), lambda k, i: (k, 0, 0)),
            pl.BlockSpec((1, HIDDEN), lambda k, i: (0, 0)),
            pl.BlockSpec((1, RB, NC), lambda k, i: (k, i, 0)),
        ],
        out_specs=pl.BlockSpec((1, RB, HIDDEN), lambda k, i: (k, i, 0)),
        out_shape=jax.ShapeDtypeStruct((3, N, HIDDEN), f32),
    )(A, W, fmT, b2, sT)


def kernel(x, edge_index, edge_weight, W, b,
           edge_mask1, feat_mask1, edge_mask2, feat_mask2):
    npad = EP - E
    pidx = (jnp.arange(npad, dtype=i32) * 13) % N
    zpad = jnp.zeros((npad,), f32)
    src2 = jnp.concatenate([edge_index[0], pidx]).reshape(NROWS, RW)
    dst2 = jnp.concatenate([edge_index[1], pidx]).reshape(NROWS, RW)
    ew2 = jnp.concatenate([edge_weight, zpad]).reshape(NROWS, RW)
    m12 = jnp.concatenate([edge_mask1.astype(f32), zpad]).reshape(NROWS, RW)
    m22 = jnp.concatenate([edge_mask2.astype(f32), zpad]).reshape(NROWS, RW)

    degpart = _deg_call(src2, dst2, ew2, m12, m22).reshape(6, NC, NPAD)
    dinv = _dinv_call(degpart)
    coef, spart = _coef_call(src2, dst2, ew2, m12, m22, dinv)

    x0 = x[:, :64]
    x1 = x[:, 64:]
    Ap1 = _agg_call(x0, x1, src2, dst2, coef, (0, 1))
    Ap2 = _agg_call(x0, x1, src2, dst2, coef, (2,))
    A = jnp.concatenate(
        [jnp.concatenate([Ap1[0], Ap2[0]], axis=0),
         jnp.concatenate([Ap1[1], Ap2[1]], axis=0)], axis=-1)

    fmT = jnp.stack([jnp.ones((D_FEAT,), f32),
                     feat_mask1.astype(f32),
                     feat_mask2.astype(f32)], axis=0).reshape(3, 1, D_FEAT)
    sT = jnp.transpose(spart.reshape(3, NC, NPAD)[:, :, :N], (0, 2, 1))
    z = _mm_call(A, W, b[None, :], fmT, sT)
    return z[0], z[1], z[2]


# speedup vs baseline: 16.9757x; 1.2874x over previous
import jax
import jax.numpy as jnp
from jax import lax
from jax.experimental import pallas as pl
from jax.experimental.pallas import tpu as pltpu
from jax.experimental.pallas import tpu_sc as plsc

N = 10000
D_FEAT = 128
HIDDEN = 128
E = 320000

NC = 2
NS = 16
NW = NC * NS
NPAD = 10240
RW = 128
EP = 327680
NROWS = EP // RW

f32 = jnp.float32
i32 = jnp.int32


def _vmesh():
    return plsc.VectorSubcoreMesh(core_axis_name="c", subcore_axis_name="s")


def _zero_vec_ref(ref, nwords):
    def body(i, _):
        ref[pl.ds(i * 16, 16)] = jnp.zeros((16,), f32)
        return 0
    lax.fori_loop(0, nwords // 16, body, 0)


def _deg_body(src_h, dst_h, ew_h, m1_h, m2_h, out_h,
              d0s, d0d, d1s, d1d, d2s, d2d,
              srcb, dstb, ewb, m1b, m2b, ew1b, ew2b, zbuf, sem):
    cid = lax.axis_index("c")
    sid = lax.axis_index("s")

    _zero_vec_ref(zbuf, 640)
    for ref in (d0s, d0d, d1s, d1d, d2s, d2d):
        pltpu.sync_copy(zbuf, ref.at[pl.ds(sid * 640, 640)])
    plsc.subcore_barrier()

    wid = cid * NS + sid
    rows_per_tile = NROWS // NW
    CH = 16

    def chunk(ch, _):
        rb = wid * rows_per_tile + ch * CH
        pltpu.sync_copy(src_h.at[pl.ds(rb, CH), :], srcb)
        pltpu.sync_copy(dst_h.at[pl.ds(rb, CH), :], dstb)
        pltpu.sync_copy(ew_h.at[pl.ds(rb, CH), :], ewb)
        pltpu.sync_copy(m1_h.at[pl.ds(rb, CH), :], m1b)
        pltpu.sync_copy(m2_h.at[pl.ds(rb, CH), :], m2b)

        def cw(r, _):
            for q in range(RW // 16):
                e = ewb[r, pl.ds(q * 16, 16)]
                ew1b[r, pl.ds(q * 16, 16)] = e * m1b[r, pl.ds(q * 16, 16)]
                ew2b[r, pl.ds(q * 16, 16)] = e * m2b[r, pl.ds(q * 16, 16)]
            return 0
        lax.fori_loop(0, CH, cw, 0)

        for g in range(4):
            cps = []
            for rr in range(4):
                r = g * 4 + rr
                for val, idx, ref in ((ewb, srcb, d0s), (ewb, dstb, d0d),
                                      (ew1b, srcb, d1s), (ew1b, dstb, d1d),
                                      (ew2b, srcb, d2s), (ew2b, dstb, d2d)):
                    cps.append(pltpu.async_copy(
                        val.at[r], ref.at[idx.at[r]], sem, add=True))
            for cp in cps:
                cp.wait()
        return 0
    lax.fori_loop(0, rows_per_tile // CH, chunk, 0)

    plsc.subcore_barrier()

    @pl.when(sid < 8)
    def _():
        for k, ref in enumerate((d0s, d0d, d1s, d1d, d2s, d2d)):
            pltpu.sync_copy(
                ref.at[pl.ds(sid * 1280, 1280)],
                out_h.at[pl.ds(k * (NC * NPAD) + cid * NPAD + sid * 1280,
                               1280)])


def _deg_call(src2, dst2, ew2, m12, m22):
    fn = pl.kernel(
        _deg_body,
        out_type=jax.ShapeDtypeStruct((6 * NC * NPAD,), f32),
        mesh=_vmesh(),
        scratch_types=[pltpu.VMEM_SHARED((NPAD,), f32) for _ in range(6)]
        + [pltpu.VMEM((16, RW), i32) for _ in range(2)]
        + [pltpu.VMEM((16, RW), f32) for _ in range(5)]
        + [pltpu.VMEM((640,), f32), pltpu.SemaphoreType.DMA],
        compiler_params=pltpu.CompilerParams(use_tc_tiling_on_sc=False),
    )
    return fn(src2, dst2, ew2, m12, m22)


def _dinv_body(deg_ref, out_ref):
    d = deg_ref[:, 0, :] + deg_ref[:, 1, :]
    out_ref[...] = lax.rsqrt(jnp.maximum(d, 1e-12))


def _dinv_call(degpart):
    return pl.pallas_call(
        _dinv_body,
        out_shape=jax.ShapeDtypeStruct((6, NPAD), f32),
    )(degpart)


def _coef_body(src_h, dst_h, ew_h, m1_h, m2_h,
               h0s, h0d, h1s, h1d, h2s, h2d,
               coef_h, sp_h,
               s0, s1, s2,
               srcb, dstb, ewb, m1b, m2b, c0b, c1b, c2b,
               g0b, g1b, g2b, g3b, g4b, g5b, zbuf, sem):
    cid = lax.axis_index("c")
    sid = lax.axis_index("s")

    _zero_vec_ref(zbuf, 640)
    for ref in (s0, s1, s2):
        pltpu.sync_copy(zbuf, ref.at[pl.ds(sid * 640, 640)])
    plsc.subcore_barrier()

    wid = cid * NS + sid
    rows_per_tile = NROWS // NW
    CH = 16

    def chunk(ch, _):
        rb = wid * rows_per_tile + ch * CH
        pltpu.sync_copy(src_h.at[pl.ds(rb, CH), :], srcb)
        pltpu.sync_copy(dst_h.at[pl.ds(rb, CH), :], dstb)
        pltpu.sync_copy(ew_h.at[pl.ds(rb, CH), :], ewb)
        pltpu.sync_copy(m1_h.at[pl.ds(rb, CH), :], m1b)
        pltpu.sync_copy(m2_h.at[pl.ds(rb, CH), :], m2b)

        cps = []
        for r in range(CH):
            for tab, idx, dstv in ((h0s, srcb, g0b), (h0d, dstb, g1b),
                                   (h1s, srcb, g2b), (h1d, dstb, g3b),
                                   (h2s, srcb, g4b), (h2d, dstb, g5b)):
                cps.append(pltpu.async_copy(
                    tab.at[idx.at[r]], dstv.at[r], sem))
        for cp in cps:
            cp.wait()

        def crow(r, _):
            for q in range(RW // 16):
                sl = pl.ds(q * 16, 16)
                ew = ewb[r, sl]
                c0b[r, sl] = ew * g0b[r, sl] * g1b[r, sl]
                c1b[r, sl] = ew * m1b[r, sl] * g2b[r, sl] * g3b[r, sl]
                c2b[r, sl] = ew * m2b[r, sl] * g4b[r, sl] * g5b[r, sl]
            return 0
        lax.fori_loop(0, CH, crow, 0)

        pltpu.sync_copy(c0b, coef_h.at[0, pl.ds(rb, CH), :])
        pltpu.sync_copy(c1b, coef_h.at[1, pl.ds(rb, CH), :])
        pltpu.sync_copy(c2b, coef_h.at[2, pl.ds(rb, CH), :])

        for g in range(4):
            cps = []
            for rr in range(4):
                r = g * 4 + rr
                for val, ref in ((c0b, s0), (c1b, s1), (c2b, s2)):
                    cps.append(pltpu.async_copy(
                        val.at[r], ref.at[dstb.at[r]], sem, add=True))
            for cp in cps:
                cp.wait()
        return 0
    lax.fori_loop(0, rows_per_tile // CH, chunk, 0)

    plsc.subcore_barrier()

    @pl.when(sid < 8)
    def _():
        for k, ref in enumerate((s0, s1, s2)):
            pltpu.sync_copy(
                ref.at[pl.ds(sid * 1280, 1280)],
                sp_h.at[pl.ds(k * (NC * NPAD) + cid * NPAD + sid * 1280,
                              1280)])


def _coef_call(src2, dst2, ew2, m12, m22, dinv):
    fn = pl.kernel(
        _coef_body,
        out_type=(jax.ShapeDtypeStruct((3, NROWS, RW), f32),
                  jax.ShapeDtypeStruct((3 * NC * NPAD,), f32)),
        mesh=_vmesh(),
        scratch_types=[pltpu.VMEM_SHARED((NPAD,), f32) for _ in range(3)]
        + [pltpu.VMEM((16, RW), i32) for _ in range(2)]
        + [pltpu.VMEM((16, RW), f32) for _ in range(12)]
        + [pltpu.VMEM((640,), f32), pltpu.SemaphoreType.DMA],
        compiler_params=pltpu.CompilerParams(use_tc_tiling_on_sc=False),
    )
    return fn(src2, dst2, ew2, m12, m22,
              dinv[0], dinv[1], dinv[2], dinv[3], dinv[4], dinv[5])


def _make_agg_body(klist):
    KN = len(klist)
    KC = KN * 64

    def _agg_body(x0_h, x1_h, src_h, dst_h, coef_h, a_h,
                  acc, srcb, dstb, c0b, c1b, xr, stg, zbuf, semg, sems):
        cid = lax.axis_index("c")
        sid = lax.axis_index("s")
        cbs = (c0b, c1b)[:KN]

        def zb(i, _):
            for q in range(KC // 16):
                zbuf[i, pl.ds(q * 16, 16)] = jnp.zeros((16,), f32)
            return 0
        lax.fori_loop(0, 8, zb, 0)

        def zrow(i, _):
            pltpu.sync_copy(zbuf, acc.at[pl.ds(sid * 640 + i * 8, 8), :])
            return 0
        lax.fori_loop(0, 80, zrow, 0)
        plsc.subcore_barrier()

        rows_per_tile = NROWS // NS
        CH = 8
        NB = CH * 2
        DEPTH = 3

        def gather(h, _ignored=None):
            r, off, slot = h // 2, (h % 2) * 64, h % DEPTH
            idx = srcb.at[r, pl.ds(off, 64)]

            @pl.when(cid == 0)
            def _():
                pltpu.async_copy(x0_h.at[idx], xr.at[slot], semg)

            @pl.when(cid == 1)
            def _():
                pltpu.async_copy(x1_h.at[idx], xr.at[slot], semg)
            return pltpu.make_async_copy(x0_h.at[idx], xr.at[slot], semg)

        def chunk(ch, _):
            rb = sid * rows_per_tile + ch * CH
            pltpu.sync_copy(src_h.at[pl.ds(rb, CH), :], srcb)
            pltpu.sync_copy(dst_h.at[pl.ds(rb, CH), :], dstb)
            for kk, cb in zip(klist, cbs):
                pltpu.sync_copy(coef_h.at[kk, pl.ds(rb, CH), :], cb)

            gw = [None] * NB
            scs = [None] * NB
            for h in range(DEPTH):
                gw[h] = gather(h)
            for h in range(NB):
                r, off, slot = h // 2, (h % 2) * 64, h % DEPTH
                gw[h].wait()
                if h >= DEPTH:
                    scs[h - DEPTH].wait()

                def egroup(g, _, r=r, off=off, slot=slot):
                    cvs = [cb[r, pl.ds(off + g * 16, 16)] for cb in cbs]
                    for li in range(16):
                        j = g * 16 + li
                        for q in range(4):
                            xq = xr[slot, j, pl.ds(q * 16, 16)]
                            for ki in range(KN):
                                stg[slot, j, pl.ds(ki * 64 + q * 16, 16)] = (
                                    cvs[ki][li] * xq)
                    return 0
                lax.fori_loop(0, 4, egroup, 0)
                scs[h] = pltpu.async_copy(
                    stg.at[slot], acc.at[dstb.at[r, pl.ds(off, 64)]],
                    sems, add=True)
                if h + DEPTH < NB:
                    gw[h + DEPTH] = gather(h + DEPTH)
            for h in range(NB - DEPTH, NB):
                scs[h].wait()
            return 0
        lax.fori_loop(0, rows_per_tile // CH, chunk, 0)

        plsc.subcore_barrier()

        @pl.when(sid < 10)
        def _():
            for ki in range(KN):
                pltpu.sync_copy(
                    acc.at[pl.ds(sid * 1000, 1000), pl.ds(ki * 64, 64)],
                    a_h.at[cid, ki, pl.ds(sid * 1000, 1000), :])
    return _agg_body


def _agg_call(x0, x1, src2, dst2, coef, klist):
    KN = len(klist)
    KC = KN * 64
    fn = pl.kernel(
        _make_agg_body(klist),
        out_type=jax.ShapeDtypeStruct((NC, KN, N, 64), f32),
        mesh=_vmesh(),
        scratch_types=[pltpu.VMEM_SHARED((NPAD, KC), f32)]
        + [pltpu.VMEM((8, RW), i32) for _ in range(2)]
        + [pltpu.VMEM((8, RW), f32) for _ in range(2)]
        + [pltpu.VMEM((3, 64, 64), f32), pltpu.VMEM((3, 64, KC), f32)]
        + [pltpu.VMEM((8, KC), f32),
           pltpu.SemaphoreType.DMA, pltpu.SemaphoreType.DMA],
        compiler_params=pltpu.CompilerParams(use_tc_tiling_on_sc=False),
    )
    return fn(x0, x1, src2, dst2, coef)


def _mm_body(a_ref, w_ref, fm_ref, b_ref, s_ref, out_ref):
    am = a_ref[0] * fm_ref[0]
    h = jnp.dot(am, w_ref[...], preferred_element_type=f32,
                precision=lax.Precision.HIGHEST)
    sv = s_ref[0, :, 0] + s_ref[0, :, 1]
    out_ref[0] = jnp.maximum(h + sv[:, None] * b_ref[...], 0.0)


def _mm_call(A, W, b2, fmT, sT):
    RB = 2000
    return pl.pallas_call(
        _mm_body,
        grid=(3, N // RB),
        in_specs=[
            pl.BlockSpec((1, RB, D_FEAT), lambda k, i: (k, i, 0)),
            pl.BlockSpec((D_FEAT, HIDDEN), lambda k, i: (0, 0)),
            pl.BlockSpec((1, 1, D_FEAT), lambda k, i: (k, 0, 0)),
            pl.BlockSpec((1, HIDDEN), lambda k, i: (0, 0)),
            pl.BlockSpec((1, RB, NC), lambda k, i: (k, i, 0)),
        ],
        out_specs=pl.BlockSpec((1, RB, HIDDEN), lambda k, i: (k, i, 0)),
        out_shape=jax.ShapeDtypeStruct((3, N, HIDDEN), f32),
    )(A, W, fmT, b2, sT)


def kernel(x, edge_index, edge_weight, W, b,
           edge_mask1, feat_mask1, edge_mask2, feat_mask2):
    npad = EP - E
    pidx = (jnp.arange(npad, dtype=i32) * 13) % N
    zpad = jnp.zeros((npad,), f32)
    src2 = jnp.concatenate([edge_index[0], pidx]).reshape(NROWS, RW)
    dst2 = jnp.concatenate([edge_index[1], pidx]).reshape(NROWS, RW)
    ew2 = jnp.concatenate([edge_weight, zpad]).reshape(NROWS, RW)
    m12 = jnp.concatenate([edge_mask1.astype(f32), zpad]).reshape(NROWS, RW)
    m22 = jnp.concatenate([edge_mask2.astype(f32), zpad]).reshape(NROWS, RW)

    degpart = _deg_call(src2, dst2, ew2, m12, m22).reshape(6, NC, NPAD)
    dinv = _dinv_call(degpart)
    coef, spart = _coef_call(src2, dst2, ew2, m12, m22, dinv)

    x0 = x[:, :64]
    x1 = x[:, 64:]
    Ap1 = _agg_call(x0, x1, src2, dst2, coef, (0, 1))
    Ap2 = _agg_call(x0, x1, src2, dst2, coef, (2,))
    A = jnp.concatenate(
        [jnp.concatenate([Ap1[0], Ap2[0]], axis=0),
         jnp.concatenate([Ap1[1], Ap2[1]], axis=0)], axis=-1)

    fmT = jnp.stack([jnp.ones((D_FEAT,), f32),
                     feat_mask1.astype(f32),
                     feat_mask2.astype(f32)], axis=0).reshape(3, 1, D_FEAT)
    sT = jnp.transpose(spart.reshape(3, NC, NPAD)[:, :, :N], (0, 2, 1))
    z = _mm_call(A, W, b[None, :], fmT, sT)
    return z[0], z[1], z[2]
